# baseline (speedup 1.0000x reference)
import functools

import jax
import jax.numpy as jnp
from jax import lax
from jax.experimental import pallas as pl

K = 16
BLOCK_ROWS = 4096
CHUNK_ROWS = 32


def _tc_body(x_ref, o_ref, *, block_rows):
    lane = lax.broadcasted_iota(jnp.int32, (CHUNK_ROWS, 128), 1)
    lane16 = lane & (K - 1)

    ri = lax.broadcasted_iota(jnp.int32, (128, 128), 0)
    ci = lax.broadcasted_iota(jnp.int32, (128, 128), 1)
    mlt = jnp.where((ri // K == ci // K) & (ri < ci), 1.0, 0.0)

    def step(i, _):
        c = x_ref[pl.ds(i * CHUNK_ROWS, CHUNK_ROWS), :]
        a = jnp.abs(c)
        m = a
        for s in (1, 2, 4, 8):
            m = jnp.maximum(m, jnp.roll(m, -s, axis=-1))
        for s in (1, 2, 4, 8):
            m = jnp.where((lane16 & s) != 0, jnp.roll(m, s, axis=-1), m)
        eq = jnp.where(a >= m, 1.0, 0.0)
        cnt = jnp.dot(eq, mlt)
        out = jnp.where((eq > 0.5) & (cnt < 0.5), c, 0.0)
        o_ref[pl.ds(i * CHUNK_ROWS, CHUNK_ROWS), :] = out
        return 0

    lax.fori_loop(0, block_rows // CHUNK_ROWS, step, 0, unroll=2)


def kernel(input_):
    b, c, l = input_.shape
    n = b * c * l // 128
    x2 = input_.reshape(n, 128)
    br = min(BLOCK_ROWS, n)
    out = pl.pallas_call(
        functools.partial(_tc_body, block_rows=br),
        grid=(n // br,),
        in_specs=[pl.BlockSpec((br, 128), lambda i: (i, 0))],
        out_specs=pl.BlockSpec((br, 128), lambda i: (i, 0)),
        out_shape=jax.ShapeDtypeStruct((n, 128), input_.dtype),
    )(x2)
    return out.reshape(b, c, l)


# speedup vs baseline: 1.5284x; 1.5284x over previous
import functools

import jax
import jax.numpy as jnp
from jax import lax
from jax.experimental import pallas as pl

K = 16
BLOCK_ROWS = 4096
CHUNK_ROWS = 16
UNROLL = 4


def _tc_body(x_ref, o_ref, *, block_rows):
    lane = lax.broadcasted_iota(jnp.int32, (CHUNK_ROWS, 128), 1)
    lane16 = lane & (K - 1)
    j16f = lane16.astype(jnp.float32)
    leader = lane16 == 0

    ri = lax.broadcasted_iota(jnp.int32, (128, 128), 0)
    ci = lax.broadcasted_iota(jnp.int32, (128, 128), 1)
    bcast = jnp.where((ri % K == 0) & (ri // K == ci // K), 1.0, 0.0)

    def step(i, _):
        c = x_ref[pl.ds(i * CHUNK_ROWS, CHUNK_ROWS), :]
        m = jnp.abs(c)
        idx = j16f
        for s in (1, 2, 4, 8):
            rm = jnp.roll(m, -s, axis=-1)
            rdx = jnp.roll(idx, -s, axis=-1)
            gt = rm > m
            m = jnp.where(gt, rm, m)
            idx = jnp.where(gt, rdx, idx)
        il = jnp.where(leader, idx, 0.0)
        g = jnp.dot(il, bcast)
        out = jnp.where(g == j16f, c, 0.0)
        o_ref[pl.ds(i * CHUNK_ROWS, CHUNK_ROWS), :] = out
        return 0

    lax.fori_loop(0, block_rows // CHUNK_ROWS, step, 0, unroll=UNROLL)


def kernel(input_):
    b, c, l = input_.shape
    n = b * c * l // 128
    x2 = input_.reshape(n, 128)
    br = min(BLOCK_ROWS, n)
    out = pl.pallas_call(
        functools.partial(_tc_body, block_rows=br),
        grid=(n // br,),
        in_specs=[pl.BlockSpec((br, 128), lambda i: (i, 0))],
        out_specs=pl.BlockSpec((br, 128), lambda i: (i, 0)),
        out_shape=jax.ShapeDtypeStruct((n, 128), input_.dtype),
    )(x2)
    return out.reshape(b, c, l)


# speedup vs baseline: 2.0000x; 1.3086x over previous
import functools

import jax
import jax.numpy as jnp
from jax import lax
from jax.experimental import pallas as pl
from jax.experimental.pallas import tpu as pltpu

K = 16
BLOCK_ROWS = 4096
CHUNK_ROWS = 8
UNROLL = 8


def _tc_body(x_ref, o_ref, il_ref, *, block_rows):
    lane_c = lax.broadcasted_iota(jnp.int32, (CHUNK_ROWS, 128), 1)
    lane16_c = lane_c & (K - 1)
    j16f_c = lane16_c.astype(jnp.float32)
    leader_c = lane16_c == 0

    ri = lax.broadcasted_iota(jnp.int32, (128, 128), 0)
    ci = lax.broadcasted_iota(jnp.int32, (128, 128), 1)
    bcast = jnp.where((ri % K == 0) & (ri // K == ci // K), 1.0, 0.0)

    def step(i, _):
        c = x_ref[pl.ds(i * CHUNK_ROWS, CHUNK_ROWS), :]
        m = jnp.abs(c)
        idx = j16f_c
        for s in (1, 2, 4, 8):
            rm = jnp.roll(m, -s, axis=-1)
            rdx = jnp.roll(idx, -s, axis=-1)
            gt = rm > m
            m = jnp.where(gt, rm, m)
            idx = jnp.where(gt, rdx, idx)
        il_ref[pl.ds(i * CHUNK_ROWS, CHUNK_ROWS), :] = jnp.where(
            leader_c, idx, 0.0)
        return 0

    lax.fori_loop(0, block_rows // CHUNK_ROWS, step, 0, unroll=UNROLL)

    lane = lax.broadcasted_iota(jnp.int32, (block_rows, 128), 1)
    j16f = (lane & (K - 1)).astype(jnp.float32)
    g = jnp.dot(il_ref[...], bcast)
    o_ref[...] = jnp.where(g == j16f, x_ref[...], 0.0)


def kernel(input_):
    b, c, l = input_.shape
    n = b * c * l // 128
    x2 = input_.reshape(n, 128)
    br = min(BLOCK_ROWS, n)
    out = pl.pallas_call(
        functools.partial(_tc_body, block_rows=br),
        grid=(n // br,),
        in_specs=[pl.BlockSpec((br, 128), lambda i: (i, 0))],
        out_specs=pl.BlockSpec((br, 128), lambda i: (i, 0)),
        out_shape=jax.ShapeDtypeStruct((n, 128), input_.dtype),
        scratch_shapes=[pltpu.VMEM((br, 128), jnp.float32)],
    )(x2)
    return out.reshape(b, c, l)


# speedup vs baseline: 5.4524x; 2.7262x over previous
import functools

import jax
import jax.numpy as jnp
from jax import lax
from jax.experimental import pallas as pl

K = 16
BLOCK_ROWS = 4096


def _tc_body(x_ref, o_ref, *, block_rows):
    lane = lax.broadcasted_iota(jnp.int32, (block_rows, 128), 1)
    lane16 = lane & (K - 1)
    j16f = lane16.astype(jnp.float32)

    ri = lax.broadcasted_iota(jnp.int32, (128, 128), 0)
    ci = lax.broadcasted_iota(jnp.int32, (128, 128), 1)
    bcast = jnp.where((ri % K == 0) & (ri // K == ci // K), 1.0, 0.0)

    c = x_ref[...]
    m = jnp.abs(c)
    idx = j16f
    for s in (1, 2, 4, 8):
        rm = jnp.roll(m, -s, axis=-1)
        rdx = jnp.roll(idx, -s, axis=-1)
        gt = rm > m
        m = jnp.where(gt, rm, m)
        idx = jnp.where(gt, rdx, idx)
    il = jnp.where(lane16 == 0, idx, 0.0)
    g = jnp.dot(il, bcast)
    o_ref[...] = jnp.where(g == j16f, c, 0.0)


def kernel(input_):
    b, c, l = input_.shape
    n = b * c * l // 128
    x2 = input_.reshape(n, 128)
    br = min(BLOCK_ROWS, n)
    out = pl.pallas_call(
        functools.partial(_tc_body, block_rows=br),
        grid=(n // br,),
        in_specs=[pl.BlockSpec((br, 128), lambda i: (i, 0))],
        out_specs=pl.BlockSpec((br, 128), lambda i: (i, 0)),
        out_shape=jax.ShapeDtypeStruct((n, 128), input_.dtype),
    )(x2)
    return out.reshape(b, c, l)
